# Initial kernel scaffold; baseline (speedup 1.0000x reference)
import jax
import jax.numpy as jnp
from jax.experimental import pallas as pl


def kernel(x, token_table, pos_table):
    raise NotImplementedError("write your pallas kernel here")


# speedup vs baseline: 1.6948x; 1.6948x over previous
import functools

import jax
import jax.numpy as jnp
from jax import lax
from jax.experimental import pallas as pl
from jax.experimental.pallas import tpu as pltpu
from jax.experimental.pallas import tpu_sc as plsc

VOCAB = 1000000
EMBED = 32
BATCH = 4096
SEQ = 200

_N = BATCH * SEQ
_NC, _NS = 2, 16
_NW = _NC * _NS
_NPW = _N // _NW
_G = 128
_CH = 1024
_NG = _CH // _G
_NCHUNK = _NPW // _CH


def _sc_body(x_hbm, tok_hbm, pos_hbm, out_hbm, idx_v, buf_t, buf_p, sem):
    wid = lax.axis_index("s") * _NC + lax.axis_index("c")
    wrow = wid * (_NPW // _G)

    def chunk(ci, _):
        grp = wrow + ci * _NG
        pltpu.sync_copy(x_hbm.at[pl.ds(grp, _NG)], idx_v)
        for j in range(_NG):
            pltpu.async_copy(tok_hbm.at[idx_v.at[j]],
                             buf_t.at[pl.ds(j * _G, _G)], sem)
        for j in range(_NG):
            pltpu.async_copy(pos_hbm.at[idx_v.at[j]],
                             buf_p.at[pl.ds(j * _G, _G)], sem)
        for j in range(_NG):
            pltpu.make_async_copy(tok_hbm.at[idx_v.at[j]],
                                  buf_t.at[pl.ds(j * _G, _G)], sem).wait()
            pltpu.make_async_copy(pos_hbm.at[idx_v.at[j]],
                                  buf_p.at[pl.ds(j * _G, _G)], sem).wait()

        @plsc.parallel_loop(0, _CH, step=1)
        def add_row(r):
            buf_t[r, pl.ds(0, 16)] += buf_p[r, pl.ds(0, 16)]
            buf_t[r, pl.ds(16, 16)] += buf_p[r, pl.ds(16, 16)]

        pltpu.sync_copy(buf_t, out_hbm.at[pl.ds(grp * _G, _CH)])
        return 0

    lax.fori_loop(0, _NCHUNK, chunk, 0)


@jax.jit
def kernel(x, token_table, pos_table):
    x2d = x.reshape(_N // _G, _G)
    mesh = plsc.VectorSubcoreMesh(core_axis_name="c", subcore_axis_name="s")
    out = pl.kernel(
        _sc_body,
        out_type=jax.ShapeDtypeStruct((_N, EMBED), jnp.float32),
        mesh=mesh,
        scratch_types=[
            pltpu.VMEM((_NG, _G), jnp.int32),
            pltpu.VMEM((_CH, EMBED), jnp.float32),
            pltpu.VMEM((_CH, EMBED), jnp.float32),
            pltpu.SemaphoreType.DMA,
        ],
        compiler_params=pltpu.CompilerParams(use_tc_tiling_on_sc=False),
    )(x2d, token_table, pos_table)
    return out.reshape(BATCH, SEQ, EMBED)


# speedup vs baseline: 1.7750x; 1.0473x over previous
import functools

import jax
import jax.numpy as jnp
from jax import lax
from jax.experimental import pallas as pl
from jax.experimental.pallas import tpu as pltpu
from jax.experimental.pallas import tpu_sc as plsc

VOCAB = 1000000
EMBED = 32
BATCH = 4096
SEQ = 200

_N = BATCH * SEQ
_NC, _NS = 2, 16
_NW = _NC * _NS
_NPW = _N // _NW
_G = 128
_CH = 1024
_NG = _CH // _G
_NCHUNK = _NPW // _CH


def _sc_body(x_hbm, tok_hbm, pos_hbm, out_hbm, idx_v, buf_t, buf_p, sem):
    wid = lax.axis_index("s") * _NC + lax.axis_index("c")
    wrow = wid * (_NPW // _G)

    def chunk(ci, _):
        grp = wrow + ci * _NG
        pltpu.sync_copy(x_hbm.at[pl.ds(grp, _NG)], idx_v)
        for j in range(_NG):
            pltpu.async_copy(tok_hbm.at[idx_v.at[j]],
                             buf_t.at[pl.ds(j * _G, _G)], sem)
        for j in range(_NG):
            pltpu.make_async_copy(tok_hbm.at[idx_v.at[j]],
                                  buf_t.at[pl.ds(j * _G, _G)], sem).wait()
        for j in range(_NG):
            pltpu.async_copy(pos_hbm.at[idx_v.at[j]],
                             buf_t.at[pl.ds(j * _G, _G)], sem, add=True)
        for j in range(_NG):
            pltpu.make_async_copy(pos_hbm.at[idx_v.at[j]],
                                  buf_t.at[pl.ds(j * _G, _G)], sem).wait()

        pltpu.sync_copy(buf_t, out_hbm.at[pl.ds(grp * _G, _CH)])
        return 0

    lax.fori_loop(0, _NCHUNK, chunk, 0)


@jax.jit
def kernel(x, token_table, pos_table):
    x2d = x.reshape(_N // _G, _G)
    mesh = plsc.VectorSubcoreMesh(core_axis_name="c", subcore_axis_name="s")
    out = pl.kernel(
        _sc_body,
        out_type=jax.ShapeDtypeStruct((_N, EMBED), jnp.float32),
        mesh=mesh,
        scratch_types=[
            pltpu.VMEM((_NG, _G), jnp.int32),
            pltpu.VMEM((_CH, EMBED), jnp.float32),
            pltpu.VMEM((_CH, EMBED), jnp.float32),
            pltpu.SemaphoreType.DMA,
        ],
        compiler_params=pltpu.CompilerParams(use_tc_tiling_on_sc=False),
    )(x2d, token_table, pos_table)
    return out.reshape(BATCH, SEQ, EMBED)


# speedup vs baseline: 1.8307x; 1.0314x over previous
import jax
import jax.numpy as jnp
from jax import lax
from jax.experimental import pallas as pl
from jax.experimental.pallas import tpu as pltpu
from jax.experimental.pallas import tpu_sc as plsc

VOCAB = 1000000
EMBED = 32
BATCH = 4096
SEQ = 200

_N = BATCH * SEQ
_NC, _NS = 2, 16
_NW = _NC * _NS
_NPW = _N // _NW
_G = 128
_CH = 1024
_NG = _CH // _G
_NCHUNK = _NPW // _CH
_NBUF = 3


def _sc_body(x_hbm, tok_hbm, pos_hbm, out_hbm, idx_v, buf, sem_t, sem_w):
    wid = lax.axis_index("s") * _NC + lax.axis_index("c")
    wrow = wid * (_NPW // _G)

    def tok_copy(g, b, j):
        return pltpu.make_async_copy(
            tok_hbm.at[idx_v.at[b, j]],
            buf.at[b, pl.ds(j * _G, _G)], sem_t.at[b])

    def pos_copy(g, b, j):
        return pltpu.make_async_copy(
            pos_hbm.at[idx_v.at[b, j]],
            buf.at[b, pl.ds(j * _G, _G)], sem_t.at[b])

    def out_copy(g, b):
        return pltpu.make_async_copy(
            buf.at[b], out_hbm.at[pl.ds((wrow + g * _NG) * _G, _CH)],
            sem_w.at[b])

    def step(g, _):
        b0 = lax.rem(g, _NBUF)
        b1 = lax.rem(g + _NBUF - 1, _NBUF)
        b2 = lax.rem(g + _NBUF - 2, _NBUF)

        @pl.when(g < _NCHUNK)
        def _stage_a():
            @pl.when(g >= _NBUF)
            def _reclaim():
                out_copy(g - _NBUF, b0).wait()
            pltpu.sync_copy(x_hbm.at[pl.ds(wrow + g * _NG, _NG)],
                            idx_v.at[b0])
            for j in range(_NG):
                tok_copy(g, b0, j).start()

        @pl.when(jnp.logical_and(g >= 1, g - 1 < _NCHUNK))
        def _stage_b():
            for j in range(_NG):
                tok_copy(g - 1, b1, j).wait()
            for j in range(_NG):
                pltpu.async_copy(pos_hbm.at[idx_v.at[b1, j]],
                                 buf.at[b1, pl.ds(j * _G, _G)],
                                 sem_t.at[b1], add=True)

        @pl.when(jnp.logical_and(g >= 2, g - 2 < _NCHUNK))
        def _stage_c():
            for j in range(_NG):
                pos_copy(g - 2, b2, j).wait()
            out_copy(g - 2, b2).start()

        return 0

    lax.fori_loop(0, _NCHUNK + 2, step, 0)
    for gg in range(max(0, _NCHUNK - _NBUF), _NCHUNK):
        out_copy(gg, gg % _NBUF).wait()


@jax.jit
def kernel(x, token_table, pos_table):
    x2d = x.reshape(_N // _G, _G)
    mesh = plsc.VectorSubcoreMesh(core_axis_name="c", subcore_axis_name="s")
    out = pl.kernel(
        _sc_body,
        out_type=jax.ShapeDtypeStruct((_N, EMBED), jnp.float32),
        mesh=mesh,
        scratch_types=[
            pltpu.VMEM((_NBUF, _NG, _G), jnp.int32),
            pltpu.VMEM((_NBUF, _CH, EMBED), jnp.float32),
            pltpu.SemaphoreType.DMA((_NBUF,)),
            pltpu.SemaphoreType.DMA((_NBUF,)),
        ],
        compiler_params=pltpu.CompilerParams(use_tc_tiling_on_sc=False),
    )(x2d, token_table, pos_table)
    return out.reshape(BATCH, SEQ, EMBED)
